# Initial kernel scaffold; baseline (speedup 1.0000x reference)
import functools

import jax
import jax.numpy as jnp
from jax import lax
from jax.experimental import pallas as pl
from jax.experimental.pallas import tpu as pltpu
from jax.experimental.pallas import tpu_sc as plsc

N = 10000
E = 320000
D = 128
BN_EPS = 1e-5

NC = 2
NS = 16
EPT = E // (NC * NS)
CHUNK = 80
NITER = EPT // CHUNK
RPT = N // NS
ZR = 125
NZ = RPT // ZR
LANES = D // 16


def _make_agg():
    mesh = plsc.VectorSubcoreMesh(core_axis_name="c", subcore_axis_name="s")

    @functools.partial(
        pl.kernel,
        out_type=jax.ShapeDtypeStruct((NC, N, D), jnp.float32),
        mesh=mesh,
        scratch_types=[
            pltpu.VMEM((CHUNK,), jnp.int32),
            pltpu.VMEM((CHUNK,), jnp.int32),
            pltpu.VMEM((CHUNK, D), jnp.float32),
            pltpu.VMEM((ZR, D), jnp.float32),
            pltpu.VMEM_SHARED((N, D), jnp.float32),
            pltpu.SemaphoreType.DMA,
        ],
    )
    def agg(h_hbm, src_hbm, dst_hbm, out_hbm, src_v, dst_v, rows_v, stage_v,
            acc_sh, sem):
        c = lax.axis_index("c")
        s = lax.axis_index("s")

        zero16 = jnp.zeros((16,), jnp.float32)

        def zfill(i, carry):
            stage_v[i // LANES, pl.ds((i % LANES) * 16, 16)] = zero16
            return carry

        lax.fori_loop(0, ZR * LANES, zfill, 0)

        r0 = s * RPT

        def zcopy(j, carry):
            pltpu.sync_copy(stage_v, acc_sh.at[pl.ds(r0 + j * ZR, ZR)])
            return carry

        lax.fori_loop(0, NZ, zcopy, 0)
        plsc.subcore_barrier()

        ebase = (c * NS + s) * EPT

        def body(i, carry):
            off = ebase + i * CHUNK
            pltpu.sync_copy(src_hbm.at[pl.ds(off, CHUNK)], src_v)
            pltpu.sync_copy(dst_hbm.at[pl.ds(off, CHUNK)], dst_v)
            pltpu.async_copy(h_hbm.at[src_v], rows_v, sem).wait()
            pltpu.sync_copy(rows_v, acc_sh.at[dst_v], add=True)
            return carry

        lax.fori_loop(0, NITER, body, 0)
        plsc.subcore_barrier()

        def ocopy(j, carry):
            sl = pl.ds(r0 + j * ZR, ZR)
            pltpu.sync_copy(acc_sh.at[sl], stage_v)
            pltpu.sync_copy(stage_v, out_hbm.at[c, sl])
            return carry

        lax.fori_loop(0, NZ, ocopy, 0)

    return agg


_agg = _make_agg()


BLK = 1000


def _mlp_body(h_ref, p0_ref, p1_ref, w1_ref, b1_ref, w2_ref, b2_ref,
              sc_ref, sh_ref, o_ref):
    z = h_ref[...] + p0_ref[...] + p1_ref[...]
    z = jnp.dot(z, w1_ref[...], preferred_element_type=jnp.float32)
    z = jnp.maximum(z + b1_ref[...], 0.0)
    z = jnp.dot(z, w2_ref[...], preferred_element_type=jnp.float32)
    z = jnp.maximum(z + b2_ref[...], 0.0)
    o_ref[...] = jnp.maximum(z * sc_ref[...] + sh_ref[...], 0.0)


_row_spec = pl.BlockSpec((BLK, D), lambda i: (i, 0))
_w_spec = pl.BlockSpec((D, D), lambda i: (0, 0))
_v_spec = pl.BlockSpec((1, D), lambda i: (0, 0))

_mlp = pl.pallas_call(
    _mlp_body,
    grid=(N // BLK,),
    in_specs=[_row_spec, _row_spec, _row_spec,
              _w_spec, _v_spec, _w_spec, _v_spec, _v_spec, _v_spec],
    out_specs=_row_spec,
    out_shape=jax.ShapeDtypeStruct((N, D), jnp.float32),
)


def kernel(x, edge_index,
           l0_W1, l0_b1, l0_W2, l0_b2, l0_gamma, l0_beta, l0_rm, l0_rv,
           l1_W1, l1_b1, l1_W2, l1_b2, l1_gamma, l1_beta, l1_rm, l1_rv,
           l2_W1, l2_b1, l2_W2, l2_b2, l2_gamma, l2_beta, l2_rm, l2_rv):
    src = edge_index[0]
    dst = edge_index[1]
    params = [
        (l0_W1, l0_b1, l0_W2, l0_b2, l0_gamma, l0_beta, l0_rm, l0_rv),
        (l1_W1, l1_b1, l1_W2, l1_b2, l1_gamma, l1_beta, l1_rm, l1_rv),
        (l2_W1, l2_b1, l2_W2, l2_b2, l2_gamma, l2_beta, l2_rm, l2_rv),
    ]
    h = x
    for (W1, b1, W2, b2, gamma, beta, rm, rv) in params:
        p = _agg(h, src, dst)
        scale = gamma * lax.rsqrt(rv + BN_EPS)
        shift = beta - rm * scale
        h = _mlp(h, p[0], p[1],
                 W1, b1.reshape(1, D), W2, b2.reshape(1, D),
                 scale.reshape(1, D), shift.reshape(1, D))
    return h


# speedup vs baseline: 4.8975x; 4.8975x over previous
import functools

import jax
import jax.numpy as jnp
from jax import lax
from jax.experimental import pallas as pl
from jax.experimental.pallas import tpu as pltpu
from jax.experimental.pallas import tpu_sc as plsc

N = 10000
E = 320000
D = 128
BN_EPS = 1e-5

NC = 2
NS = 16
EPT = E // (NC * NS)
CHUNK = 80
NITER = EPT // CHUNK
NP = 10240
RPT = NP // NS
ZR = 128
NZ = RPT // ZR
LANES = D // 16


def _make_agg():
    mesh = plsc.VectorSubcoreMesh(core_axis_name="c", subcore_axis_name="s")

    @functools.partial(
        pl.kernel,
        out_type=jax.ShapeDtypeStruct((NC, NP, D), jnp.float32),
        mesh=mesh,
        scratch_types=[
            pltpu.VMEM((CHUNK,), jnp.int32),
            pltpu.VMEM((CHUNK,), jnp.int32),
            pltpu.VMEM((CHUNK, D), jnp.float32),
            pltpu.VMEM((ZR, D), jnp.float32),
            pltpu.VMEM_SHARED((NP, D), jnp.float32),
            pltpu.SemaphoreType.DMA,
        ],
    )
    def agg(h_hbm, src_hbm, dst_hbm, out_hbm, src_v, dst_v, rows_v, stage_v,
            acc_sh, sem):
        c = lax.axis_index("c")
        s = lax.axis_index("s")

        zero16 = jnp.zeros((16,), jnp.float32)

        def zfill(i, carry):
            stage_v[i // LANES, pl.ds((i % LANES) * 16, 16)] = zero16
            return carry

        lax.fori_loop(0, ZR * LANES, zfill, 0)

        r0 = s * RPT

        def zcopy(j, carry):
            pltpu.sync_copy(stage_v, acc_sh.at[pl.ds(r0 + j * ZR, ZR)])
            return carry

        lax.fori_loop(0, NZ, zcopy, 0)
        plsc.subcore_barrier()

        ebase = (c * NS + s) * EPT

        def body(i, carry):
            off = ebase + i * CHUNK
            pltpu.sync_copy(src_hbm.at[pl.ds(off, CHUNK)], src_v)
            pltpu.sync_copy(dst_hbm.at[pl.ds(off, CHUNK)], dst_v)
            pltpu.async_copy(h_hbm.at[src_v], rows_v, sem).wait()
            pltpu.sync_copy(rows_v, acc_sh.at[dst_v], add=True)
            return carry

        lax.fori_loop(0, NITER, body, 0)
        plsc.subcore_barrier()

        def ocopy(j, carry):
            sl = pl.ds(r0 + j * ZR, ZR)
            pltpu.sync_copy(acc_sh.at[sl], stage_v)
            pltpu.sync_copy(stage_v, out_hbm.at[c, sl])
            return carry

        lax.fori_loop(0, NZ, ocopy, 0)

    return agg


_agg = _make_agg()


BLK = 1000


def _mlp_body(h_ref, p0_ref, p1_ref, w1_ref, b1_ref, w2_ref, b2_ref,
              sc_ref, sh_ref, o_ref):
    z = h_ref[...] + p0_ref[...] + p1_ref[...]
    z = jnp.dot(z, w1_ref[...], preferred_element_type=jnp.float32)
    z = jnp.maximum(z + b1_ref[...], 0.0)
    z = jnp.dot(z, w2_ref[...], preferred_element_type=jnp.float32)
    z = jnp.maximum(z + b2_ref[...], 0.0)
    o_ref[...] = jnp.maximum(z * sc_ref[...] + sh_ref[...], 0.0)


_row_spec = pl.BlockSpec((BLK, D), lambda i: (i, 0))
_w_spec = pl.BlockSpec((D, D), lambda i: (0, 0))
_v_spec = pl.BlockSpec((1, D), lambda i: (0, 0))

_mlp = pl.pallas_call(
    _mlp_body,
    grid=(N // BLK,),
    in_specs=[_row_spec, _row_spec, _row_spec,
              _w_spec, _v_spec, _w_spec, _v_spec, _v_spec, _v_spec],
    out_specs=_row_spec,
    out_shape=jax.ShapeDtypeStruct((N, D), jnp.float32),
)


def kernel(x, edge_index,
           l0_W1, l0_b1, l0_W2, l0_b2, l0_gamma, l0_beta, l0_rm, l0_rv,
           l1_W1, l1_b1, l1_W2, l1_b2, l1_gamma, l1_beta, l1_rm, l1_rv,
           l2_W1, l2_b1, l2_W2, l2_b2, l2_gamma, l2_beta, l2_rm, l2_rv):
    src = edge_index[0]
    dst = edge_index[1]
    params = [
        (l0_W1, l0_b1, l0_W2, l0_b2, l0_gamma, l0_beta, l0_rm, l0_rv),
        (l1_W1, l1_b1, l1_W2, l1_b2, l1_gamma, l1_beta, l1_rm, l1_rv),
        (l2_W1, l2_b1, l2_W2, l2_b2, l2_gamma, l2_beta, l2_rm, l2_rv),
    ]
    h = x
    for (W1, b1, W2, b2, gamma, beta, rm, rv) in params:
        p = _agg(h, src, dst)[:, :N, :]
        scale = gamma * lax.rsqrt(rv + BN_EPS)
        shift = beta - rm * scale
        h = _mlp(h, p[0], p[1],
                 W1, b1.reshape(1, D), W2, b2.reshape(1, D),
                 scale.reshape(1, D), shift.reshape(1, D))
    return h


# speedup vs baseline: 7.8055x; 1.5938x over previous
import functools

import jax
import jax.numpy as jnp
from jax import lax
from jax.experimental import pallas as pl
from jax.experimental.pallas import tpu as pltpu
from jax.experimental.pallas import tpu_sc as plsc

N = 10000
E = 320000
D = 128
BN_EPS = 1e-5

NC = 2
NS = 16
NW = NC * NS
EPT = E // NW
CHUNK = 80
NITER = EPT // CHUNK
NPAIR = (NITER - 1) // 2
NP = 10240
RPT = NP // NS
ZR = 128
NZ = RPT // ZR
LANES = D // 16


def _make_agg():
    mesh = plsc.VectorSubcoreMesh(core_axis_name="c", subcore_axis_name="s")

    @functools.partial(
        pl.kernel,
        out_type=jax.ShapeDtypeStruct((NC, NP, D), jnp.float32),
        mesh=mesh,
        scratch_types=[
            pltpu.VMEM((2, CHUNK), jnp.int32),
            pltpu.VMEM((2, CHUNK), jnp.int32),
            pltpu.VMEM((CHUNK, D), jnp.float32),
            pltpu.VMEM((CHUNK, D), jnp.float32),
            pltpu.VMEM((ZR, D), jnp.float32),
            pltpu.VMEM_SHARED((NP, D), jnp.float32),
            pltpu.SemaphoreType.DMA,
        ],
    )
    def agg(h_hbm, src_hbm, dst_hbm, out_hbm, sidx, didx, rows0, rows1,
            stage_v, acc_sh, sem):
        c = lax.axis_index("c")
        s = lax.axis_index("s")
        wid = c * NS + s
        ebase = wid * EPT

        def load_idx(i, slot):
            off = ebase + i * CHUNK
            pltpu.sync_copy(src_hbm.at[pl.ds(off, CHUNK)], sidx.at[slot])
            pltpu.sync_copy(dst_hbm.at[pl.ds(off, CHUNK)], didx.at[slot])

        zero16 = jnp.zeros((16,), jnp.float32)

        def zfill(i, carry):
            stage_v[i // LANES, pl.ds((i % LANES) * 16, 16)] = zero16
            return carry

        lax.fori_loop(0, ZR * LANES, zfill, 0)

        r0 = s * RPT

        def zcopy(j, carry):
            pltpu.sync_copy(stage_v, acc_sh.at[pl.ds(r0 + j * ZR, ZR)])
            return carry

        lax.fori_loop(0, NZ, zcopy, 0)
        plsc.subcore_barrier()

        load_idx(0, 0)
        pltpu.async_copy(h_hbm.at[sidx.at[0]], rows0, sem)
        load_idx(1, 1)

        def body(k, carry):
            i0 = 2 * k
            pltpu.make_async_copy(h_hbm.at[sidx.at[0]], rows0, sem).wait()
            pltpu.async_copy(h_hbm.at[sidx.at[1]], rows1, sem)
            pltpu.sync_copy(rows0, acc_sh.at[didx.at[0]], add=True)
            load_idx(i0 + 2, 0)
            pltpu.make_async_copy(h_hbm.at[sidx.at[1]], rows1, sem).wait()
            pltpu.async_copy(h_hbm.at[sidx.at[0]], rows0, sem)
            pltpu.sync_copy(rows1, acc_sh.at[didx.at[1]], add=True)

            @pl.when(i0 + 3 < NITER)
            def _():
                load_idx(i0 + 3, 1)

            return carry

        lax.fori_loop(0, NPAIR, body, 0)
        pltpu.make_async_copy(h_hbm.at[sidx.at[0]], rows0, sem).wait()
        pltpu.sync_copy(rows0, acc_sh.at[didx.at[0]], add=True)
        plsc.subcore_barrier()

        def ocopy(j, carry):
            sl = pl.ds(r0 + j * ZR, ZR)
            pltpu.sync_copy(acc_sh.at[sl], stage_v)
            pltpu.sync_copy(stage_v, out_hbm.at[c, sl])
            return carry

        lax.fori_loop(0, NZ, ocopy, 0)

    return agg


_agg = _make_agg()


BLK = 1000


def _mlp_body(h_ref, p0_ref, p1_ref, w1_ref, b1_ref, w2_ref, b2_ref,
              sc_ref, sh_ref, o_ref):
    z = h_ref[...] + p0_ref[...] + p1_ref[...]
    z = jnp.dot(z, w1_ref[...], preferred_element_type=jnp.float32)
    z = jnp.maximum(z + b1_ref[...], 0.0)
    z = jnp.dot(z, w2_ref[...], preferred_element_type=jnp.float32)
    z = jnp.maximum(z + b2_ref[...], 0.0)
    o_ref[...] = jnp.maximum(z * sc_ref[...] + sh_ref[...], 0.0)


_row_spec = pl.BlockSpec((BLK, D), lambda i: (i, 0))
_w_spec = pl.BlockSpec((D, D), lambda i: (0, 0))
_v_spec = pl.BlockSpec((1, D), lambda i: (0, 0))

_mlp = pl.pallas_call(
    _mlp_body,
    grid=(N // BLK,),
    in_specs=[_row_spec, _row_spec, _row_spec,
              _w_spec, _v_spec, _w_spec, _v_spec, _v_spec, _v_spec],
    out_specs=_row_spec,
    out_shape=jax.ShapeDtypeStruct((N, D), jnp.float32),
)


def kernel(x, edge_index,
           l0_W1, l0_b1, l0_W2, l0_b2, l0_gamma, l0_beta, l0_rm, l0_rv,
           l1_W1, l1_b1, l1_W2, l1_b2, l1_gamma, l1_beta, l1_rm, l1_rv,
           l2_W1, l2_b1, l2_W2, l2_b2, l2_gamma, l2_beta, l2_rm, l2_rv):
    src = edge_index[0]
    dst = edge_index[1]
    params = [
        (l0_W1, l0_b1, l0_W2, l0_b2, l0_gamma, l0_beta, l0_rm, l0_rv),
        (l1_W1, l1_b1, l1_W2, l1_b2, l1_gamma, l1_beta, l1_rm, l1_rv),
        (l2_W1, l2_b1, l2_W2, l2_b2, l2_gamma, l2_beta, l2_rm, l2_rv),
    ]
    h = x
    for (W1, b1, W2, b2, gamma, beta, rm, rv) in params:
        p = _agg(h, src, dst)[:, :N, :]
        scale = gamma * lax.rsqrt(rv + BN_EPS)
        shift = beta - rm * scale
        h = _mlp(h, p[0], p[1],
                 W1, b1.reshape(1, D), W2, b2.reshape(1, D),
                 scale.reshape(1, D), shift.reshape(1, D))
    return h


# speedup vs baseline: 8.9496x; 1.1466x over previous
import functools

import jax
import jax.numpy as jnp
from jax import lax
from jax.experimental import pallas as pl
from jax.experimental.pallas import tpu as pltpu
from jax.experimental.pallas import tpu_sc as plsc

N = 10000
E = 320000
D = 128
BN_EPS = 1e-5

NC = 2
NS = 16
NW = NC * NS
EPT = E // NW
CHUNK = 80
NITER = EPT // CHUNK
NPAIR = (NITER - 1) // 2
NP = 10240
RPT = NP // NS
ZR = CHUNK
NZ = RPT // ZR
LANES = D // 16


def _make_agg():
    mesh = plsc.VectorSubcoreMesh(core_axis_name="c", subcore_axis_name="s")

    @functools.partial(
        pl.kernel,
        out_type=jax.ShapeDtypeStruct((NC, NP, D), jnp.float32),
        mesh=mesh,
        scratch_types=[
            pltpu.VMEM((EPT,), jnp.int32),
            pltpu.VMEM((EPT,), jnp.int32),
            pltpu.VMEM((2, CHUNK), jnp.int32),
            pltpu.VMEM((CHUNK, D), jnp.float32),
            pltpu.VMEM((CHUNK, D), jnp.float32),
            pltpu.VMEM_SHARED((NP, D), jnp.float32),
            pltpu.SemaphoreType.DMA,
        ],
    )
    def agg(h_hbm, src_hbm, dst_hbm, out_hbm, sidx_all, didx_all, dbuf,
            rows0, rows1, acc_sh, sem):
        c = lax.axis_index("c")
        s = lax.axis_index("s")
        wid = c * NS + s
        ebase = wid * EPT

        def sslice(i):
            return sidx_all.at[pl.ds(i * CHUNK, CHUNK)]

        def copy_didx(i, slot):
            base = i * CHUNK
            for j in range(CHUNK // 16):
                dbuf[slot, pl.ds(j * 16, 16)] = didx_all[pl.ds(base + j * 16, 16)]

        zero16 = jnp.zeros((16,), jnp.float32)

        def zfill(i, carry):
            rows0[i // LANES, pl.ds((i % LANES) * 16, 16)] = zero16
            return carry

        lax.fori_loop(0, ZR * LANES, zfill, 0)

        r0 = s * RPT

        def zcopy(j, carry):
            pltpu.sync_copy(rows0, acc_sh.at[pl.ds(r0 + j * ZR, ZR)])
            return carry

        lax.fori_loop(0, NZ, zcopy, 0)
        plsc.subcore_barrier()

        pltpu.sync_copy(src_hbm.at[pl.ds(ebase, EPT)], sidx_all)
        pltpu.sync_copy(dst_hbm.at[pl.ds(ebase, EPT)], didx_all)

        pltpu.async_copy(h_hbm.at[sslice(0)], rows0, sem)
        copy_didx(0, 0)
        copy_didx(1, 1)

        def body(k, carry):
            i0 = 2 * k
            pltpu.make_async_copy(h_hbm.at[sslice(i0)], rows0, sem).wait()
            pltpu.async_copy(h_hbm.at[sslice(i0 + 1)], rows1, sem)
            pltpu.sync_copy(rows0, acc_sh.at[dbuf.at[0]], add=True)
            copy_didx(i0 + 2, 0)
            pltpu.make_async_copy(h_hbm.at[sslice(i0 + 1)], rows1, sem).wait()
            pltpu.async_copy(h_hbm.at[sslice(i0 + 2)], rows0, sem)
            pltpu.sync_copy(rows1, acc_sh.at[dbuf.at[1]], add=True)

            @pl.when(i0 + 3 < NITER)
            def _():
                copy_didx(i0 + 3, 1)

            return carry

        lax.fori_loop(0, NPAIR, body, 0)
        pltpu.make_async_copy(h_hbm.at[sslice(NITER - 1)], rows0, sem).wait()
        pltpu.sync_copy(rows0, acc_sh.at[dbuf.at[0]], add=True)
        plsc.subcore_barrier()

        def ocopy(j, carry):
            sl = pl.ds(r0 + j * ZR, ZR)
            pltpu.sync_copy(acc_sh.at[sl], rows0)
            pltpu.sync_copy(rows0, out_hbm.at[c, sl])
            return carry

        lax.fori_loop(0, NZ, ocopy, 0)

    return agg


_agg = _make_agg()


BLK = 1000


def _mlp_body(h_ref, p0_ref, p1_ref, w1_ref, b1_ref, w2_ref, b2_ref,
              sc_ref, sh_ref, o_ref):
    z = h_ref[...] + p0_ref[...] + p1_ref[...]
    z = jnp.dot(z, w1_ref[...], preferred_element_type=jnp.float32)
    z = jnp.maximum(z + b1_ref[...], 0.0)
    z = jnp.dot(z, w2_ref[...], preferred_element_type=jnp.float32)
    z = jnp.maximum(z + b2_ref[...], 0.0)
    o_ref[...] = jnp.maximum(z * sc_ref[...] + sh_ref[...], 0.0)


_row_spec = pl.BlockSpec((BLK, D), lambda i: (i, 0))
_w_spec = pl.BlockSpec((D, D), lambda i: (0, 0))
_v_spec = pl.BlockSpec((1, D), lambda i: (0, 0))

_mlp = pl.pallas_call(
    _mlp_body,
    grid=(N // BLK,),
    in_specs=[_row_spec, _row_spec, _row_spec,
              _w_spec, _v_spec, _w_spec, _v_spec, _v_spec, _v_spec],
    out_specs=_row_spec,
    out_shape=jax.ShapeDtypeStruct((N, D), jnp.float32),
)


def kernel(x, edge_index,
           l0_W1, l0_b1, l0_W2, l0_b2, l0_gamma, l0_beta, l0_rm, l0_rv,
           l1_W1, l1_b1, l1_W2, l1_b2, l1_gamma, l1_beta, l1_rm, l1_rv,
           l2_W1, l2_b1, l2_W2, l2_b2, l2_gamma, l2_beta, l2_rm, l2_rv):
    src = edge_index[0]
    dst = edge_index[1]
    params = [
        (l0_W1, l0_b1, l0_W2, l0_b2, l0_gamma, l0_beta, l0_rm, l0_rv),
        (l1_W1, l1_b1, l1_W2, l1_b2, l1_gamma, l1_beta, l1_rm, l1_rv),
        (l2_W1, l2_b1, l2_W2, l2_b2, l2_gamma, l2_beta, l2_rm, l2_rv),
    ]
    h = x
    for (W1, b1, W2, b2, gamma, beta, rm, rv) in params:
        p = _agg(h, src, dst)[:, :N, :]
        scale = gamma * lax.rsqrt(rv + BN_EPS)
        shift = beta - rm * scale
        h = _mlp(h, p[0], p[1],
                 W1, b1.reshape(1, D), W2, b2.reshape(1, D),
                 scale.reshape(1, D), shift.reshape(1, D))
    return h


# speedup vs baseline: 11.2387x; 1.2558x over previous
import functools

import jax
import jax.numpy as jnp
from jax import lax
from jax.experimental import pallas as pl
from jax.experimental.pallas import tpu as pltpu
from jax.experimental.pallas import tpu_sc as plsc

N = 10000
E = 320000
D = 128
BN_EPS = 1e-5

NC = 2
NS = 16
NW = NC * NS
EPT = E // NW
CHUNK = 80
NITER = EPT // CHUNK
NBUF = 4
NQUAD = (NITER - 1) // NBUF
IBYTES = CHUNK * 4
NP = 10240
RPT = NP // NS
ZR = CHUNK
NZ = RPT // ZR
LANES = D // 16


def _make_agg():
    mesh = plsc.VectorSubcoreMesh(core_axis_name="c", subcore_axis_name="s")

    @functools.partial(
        pl.kernel,
        out_type=jax.ShapeDtypeStruct((NC, NP, D), jnp.float32),
        mesh=mesh,
        scratch_types=[
            pltpu.VMEM((NBUF, CHUNK), jnp.int32),
            pltpu.VMEM((NBUF, CHUNK), jnp.int32),
            [pltpu.VMEM((CHUNK, D), jnp.float32) for _ in range(NBUF)],
            pltpu.VMEM_SHARED((NP, D), jnp.float32),
            pltpu.SemaphoreType.DMA,
            pltpu.SemaphoreType.DMA,
            pltpu.SemaphoreType.DMA,
        ],
    )
    def agg(h_hbm, src_hbm, dst_hbm, out_hbm, sbuf, dbuf, rows,
            acc_sh, isem, gsem, ssem):
        c = lax.axis_index("c")
        s = lax.axis_index("s")
        wid = c * NS + s
        ebase = wid * EPT

        def idx_load(i, b):
            off = ebase + i * CHUNK
            pltpu.async_copy(src_hbm.at[pl.ds(off, CHUNK)], sbuf.at[b], isem)
            pltpu.async_copy(dst_hbm.at[pl.ds(off, CHUNK)], dbuf.at[b], isem)

        def idx_wait(i, b):
            off = ebase + i * CHUNK
            pltpu.make_async_copy(
                src_hbm.at[pl.ds(off, CHUNK)], sbuf.at[b], isem).wait()
            pltpu.make_async_copy(
                dst_hbm.at[pl.ds(off, CHUNK)], dbuf.at[b], isem).wait()

        def g_issue(b):
            pltpu.async_copy(h_hbm.at[sbuf.at[b]], rows[b], gsem)

        def g_wait(b):
            pltpu.make_async_copy(h_hbm.at[sbuf.at[b]], rows[b], gsem).wait()

        def s_issue(b):
            pltpu.async_copy(rows[b], acc_sh.at[dbuf.at[b]], ssem, add=True)

        def s_wait(b):
            pltpu.make_async_copy(rows[b], acc_sh.at[dbuf.at[b]], ssem).wait()

        for b in range(NBUF):
            idx_load(b, b)

        zero16 = jnp.zeros((16,), jnp.float32)

        def zfill(i, carry):
            rows[0][i // LANES, pl.ds((i % LANES) * 16, 16)] = zero16
            return carry

        lax.fori_loop(0, ZR * LANES, zfill, 0)

        r0 = s * RPT

        def zcopy(j, carry):
            pltpu.sync_copy(rows[0], acc_sh.at[pl.ds(r0 + j * ZR, ZR)])
            return carry

        lax.fori_loop(0, NZ, zcopy, 0)
        plsc.subcore_barrier()

        for b in range(NBUF):
            idx_wait(b, b)
            g_issue(b)

        def body(k, carry):
            i0 = NBUF * k
            for b in range(NBUF):
                g_wait(b)
                s_issue(b)
            for b in range(NBUF):
                i = i0 + b
                inext = i + NBUF
                s_wait(b)

                @pl.when(inext < NITER)
                def _():
                    idx_load(inext, b)
                    idx_wait(inext, b)
                    g_issue(b)

            return carry

        lax.fori_loop(0, NQUAD, body, 0)
        g_wait(0)
        s_issue(0)
        s_wait(0)
        plsc.subcore_barrier()

        def ocopy(j, carry):
            sl = pl.ds(r0 + j * ZR, ZR)
            pltpu.sync_copy(acc_sh.at[sl], rows[0])
            pltpu.sync_copy(rows[0], out_hbm.at[c, sl])
            return carry

        lax.fori_loop(0, NZ, ocopy, 0)

    return agg


_agg = _make_agg()


BLK = 1000


def _mlp_body(h_ref, p0_ref, p1_ref, w1_ref, b1_ref, w2_ref, b2_ref,
              sc_ref, sh_ref, o_ref):
    z = h_ref[...] + p0_ref[...] + p1_ref[...]
    z = jnp.dot(z, w1_ref[...], preferred_element_type=jnp.float32)
    z = jnp.maximum(z + b1_ref[...], 0.0)
    z = jnp.dot(z, w2_ref[...], preferred_element_type=jnp.float32)
    z = jnp.maximum(z + b2_ref[...], 0.0)
    o_ref[...] = jnp.maximum(z * sc_ref[...] + sh_ref[...], 0.0)


_row_spec = pl.BlockSpec((BLK, D), lambda i: (i, 0))
_w_spec = pl.BlockSpec((D, D), lambda i: (0, 0))
_v_spec = pl.BlockSpec((1, D), lambda i: (0, 0))

_mlp = pl.pallas_call(
    _mlp_body,
    grid=(N // BLK,),
    in_specs=[_row_spec, _row_spec, _row_spec,
              _w_spec, _v_spec, _w_spec, _v_spec, _v_spec, _v_spec],
    out_specs=_row_spec,
    out_shape=jax.ShapeDtypeStruct((N, D), jnp.float32),
)


def kernel(x, edge_index,
           l0_W1, l0_b1, l0_W2, l0_b2, l0_gamma, l0_beta, l0_rm, l0_rv,
           l1_W1, l1_b1, l1_W2, l1_b2, l1_gamma, l1_beta, l1_rm, l1_rv,
           l2_W1, l2_b1, l2_W2, l2_b2, l2_gamma, l2_beta, l2_rm, l2_rv):
    src = edge_index[0]
    dst = edge_index[1]
    params = [
        (l0_W1, l0_b1, l0_W2, l0_b2, l0_gamma, l0_beta, l0_rm, l0_rv),
        (l1_W1, l1_b1, l1_W2, l1_b2, l1_gamma, l1_beta, l1_rm, l1_rv),
        (l2_W1, l2_b1, l2_W2, l2_b2, l2_gamma, l2_beta, l2_rm, l2_rv),
    ]
    h = x
    for (W1, b1, W2, b2, gamma, beta, rm, rv) in params:
        p = _agg(h, src, dst)[:, :N, :]
        scale = gamma * lax.rsqrt(rv + BN_EPS)
        shift = beta - rm * scale
        h = _mlp(h, p[0], p[1],
                 W1, b1.reshape(1, D), W2, b2.reshape(1, D),
                 scale.reshape(1, D), shift.reshape(1, D))
    return h


# speedup vs baseline: 11.3362x; 1.0087x over previous
import functools

import jax
import jax.numpy as jnp
from jax import lax
from jax.experimental import pallas as pl
from jax.experimental.pallas import tpu as pltpu
from jax.experimental.pallas import tpu_sc as plsc

N = 10000
E = 320000
D = 128
BN_EPS = 1e-5

NC = 2
NS = 16
NW = NC * NS
EPT = E // NW
CHUNK = 80
NITER = EPT // CHUNK
NBUF = 4
NQUAD = (NITER - 1) // NBUF
IBYTES = CHUNK * 4
NP = 10240
RPT = NP // NS
ZR = CHUNK
NZ = RPT // ZR
LANES = D // 16


def _make_agg():
    mesh = plsc.VectorSubcoreMesh(core_axis_name="c", subcore_axis_name="s")

    @functools.partial(
        pl.kernel,
        out_type=jax.ShapeDtypeStruct((NC, NP, D), jnp.float32),
        mesh=mesh,
        scratch_types=[
            pltpu.VMEM((NBUF, CHUNK), jnp.int32),
            pltpu.VMEM((NBUF, CHUNK), jnp.int32),
            [pltpu.VMEM((CHUNK, D), jnp.float32) for _ in range(NBUF)],
            pltpu.VMEM_SHARED((NP, D), jnp.float32),
            pltpu.SemaphoreType.DMA,
            pltpu.SemaphoreType.DMA,
            pltpu.SemaphoreType.DMA,
        ],
    )
    def agg(h_hbm, src_hbm, dst_hbm, out_hbm, sbuf, dbuf, rows,
            acc_sh, isem, gsem, ssem):
        c = lax.axis_index("c")
        s = lax.axis_index("s")
        wid = c * NS + s
        ebase = wid * EPT

        def idx_load(i, b):
            off = ebase + i * CHUNK
            pltpu.async_copy(src_hbm.at[pl.ds(off, CHUNK)], sbuf.at[b], isem)
            pltpu.async_copy(dst_hbm.at[pl.ds(off, CHUNK)], dbuf.at[b], isem)

        def idx_wait(i, b):
            off = ebase + i * CHUNK
            pltpu.make_async_copy(
                src_hbm.at[pl.ds(off, CHUNK)], sbuf.at[b], isem).wait()
            pltpu.make_async_copy(
                dst_hbm.at[pl.ds(off, CHUNK)], dbuf.at[b], isem).wait()

        def g_issue(b):
            pltpu.async_copy(h_hbm.at[sbuf.at[b]], rows[b], gsem)

        def g_wait(b):
            pltpu.make_async_copy(h_hbm.at[sbuf.at[b]], rows[b], gsem).wait()

        def s_issue(b):
            pltpu.async_copy(rows[b], acc_sh.at[dbuf.at[b]], ssem, add=True)

        def s_wait(b):
            pltpu.make_async_copy(rows[b], acc_sh.at[dbuf.at[b]], ssem).wait()

        for b in range(NBUF):
            idx_load(b, b)

        zero16 = jnp.zeros((16,), jnp.float32)

        def zfill(i, carry):
            rows[0][i // LANES, pl.ds((i % LANES) * 16, 16)] = zero16
            return carry

        lax.fori_loop(0, ZR * LANES, zfill, 0)

        r0 = s * RPT

        def zcopy(j, carry):
            pltpu.sync_copy(rows[0], acc_sh.at[pl.ds(r0 + j * ZR, ZR)])
            return carry

        lax.fori_loop(0, NZ, zcopy, 0)
        plsc.subcore_barrier()

        for b in range(NBUF):
            idx_wait(b, b)
            g_issue(b)

        def body(k, carry):
            i0 = NBUF * k
            for b in range(NBUF):
                g_wait(b)
                s_issue(b)
            for b in range(NBUF):
                i = i0 + b
                inext = i + NBUF
                s_wait(b)

                @pl.when(inext < NITER)
                def _():
                    idx_load(inext, b)
                    idx_wait(inext, b)
                    g_issue(b)

            return carry

        lax.fori_loop(0, NQUAD, body, 0)
        g_wait(0)
        s_issue(0)
        s_wait(0)
        plsc.subcore_barrier()

        def ocopy(j, carry):
            sl = pl.ds(r0 + j * ZR, ZR)
            pltpu.sync_copy(acc_sh.at[sl], out_hbm.at[c, sl])
            return carry

        lax.fori_loop(0, NZ, ocopy, 0)

    return agg


_agg = _make_agg()


BLK = 1000


def _mlp_body(h_ref, p0_ref, p1_ref, w1_ref, b1_ref, w2_ref, b2_ref,
              sc_ref, sh_ref, o_ref):
    z = h_ref[...] + p0_ref[...] + p1_ref[...]
    z = jnp.dot(z, w1_ref[...], preferred_element_type=jnp.float32)
    z = jnp.maximum(z + b1_ref[...], 0.0)
    z = jnp.dot(z, w2_ref[...], preferred_element_type=jnp.float32)
    z = jnp.maximum(z + b2_ref[...], 0.0)
    o_ref[...] = jnp.maximum(z * sc_ref[...] + sh_ref[...], 0.0)


_row_spec = pl.BlockSpec((BLK, D), lambda i: (i, 0))
_w_spec = pl.BlockSpec((D, D), lambda i: (0, 0))
_v_spec = pl.BlockSpec((1, D), lambda i: (0, 0))

_mlp = pl.pallas_call(
    _mlp_body,
    grid=(N // BLK,),
    in_specs=[_row_spec, _row_spec, _row_spec,
              _w_spec, _v_spec, _w_spec, _v_spec, _v_spec, _v_spec],
    out_specs=_row_spec,
    out_shape=jax.ShapeDtypeStruct((N, D), jnp.float32),
)


def kernel(x, edge_index,
           l0_W1, l0_b1, l0_W2, l0_b2, l0_gamma, l0_beta, l0_rm, l0_rv,
           l1_W1, l1_b1, l1_W2, l1_b2, l1_gamma, l1_beta, l1_rm, l1_rv,
           l2_W1, l2_b1, l2_W2, l2_b2, l2_gamma, l2_beta, l2_rm, l2_rv):
    src = edge_index[0]
    dst = edge_index[1]
    params = [
        (l0_W1, l0_b1, l0_W2, l0_b2, l0_gamma, l0_beta, l0_rm, l0_rv),
        (l1_W1, l1_b1, l1_W2, l1_b2, l1_gamma, l1_beta, l1_rm, l1_rv),
        (l2_W1, l2_b1, l2_W2, l2_b2, l2_gamma, l2_beta, l2_rm, l2_rv),
    ]
    h = x
    for (W1, b1, W2, b2, gamma, beta, rm, rv) in params:
        p = _agg(h, src, dst)[:, :N, :]
        scale = gamma * lax.rsqrt(rv + BN_EPS)
        shift = beta - rm * scale
        h = _mlp(h, p[0], p[1],
                 W1, b1.reshape(1, D), W2, b2.reshape(1, D),
                 scale.reshape(1, D), shift.reshape(1, D))
    return h


# speedup vs baseline: 11.5533x; 1.0192x over previous
import functools

import jax
import jax.numpy as jnp
from jax import lax
from jax.experimental import pallas as pl
from jax.experimental.pallas import tpu as pltpu
from jax.experimental.pallas import tpu_sc as plsc

N = 10000
E = 320000
D = 128
BN_EPS = 1e-5

NC = 2
NS = 16
NW = NC * NS
EPT = E // NW
CHUNK = 80
NITER = EPT // CHUNK
NBUF = 4
NQUAD = (NITER - 1) // NBUF
IBYTES = CHUNK * 4
NP = 10240
RPT = NP // NS
ZR = CHUNK
NZ = RPT // ZR
LANES = D // 16


def _make_agg():
    mesh = plsc.VectorSubcoreMesh(core_axis_name="c", subcore_axis_name="s")

    @functools.partial(
        pl.kernel,
        out_type=jax.ShapeDtypeStruct((NC, NP, D), jnp.float32),
        mesh=mesh,
        scratch_types=[
            pltpu.VMEM((NBUF, CHUNK), jnp.int32),
            pltpu.VMEM((NBUF, CHUNK), jnp.int32),
            [pltpu.VMEM((CHUNK, D), jnp.float32) for _ in range(NBUF)],
            pltpu.VMEM_SHARED((NP, D), jnp.float32),
            pltpu.SemaphoreType.DMA,
            pltpu.SemaphoreType.DMA,
            pltpu.SemaphoreType.DMA,
        ],
    )
    def agg(h_hbm, src_hbm, dst_hbm, out_hbm, sbuf, dbuf, rows,
            acc_sh, isem, gsem, ssem):
        c = lax.axis_index("c")
        s = lax.axis_index("s")
        wid = c * NS + s
        ebase = wid * EPT

        def idx_load(i, b):
            off = ebase + i * CHUNK
            pltpu.async_copy(src_hbm.at[pl.ds(off, CHUNK)], sbuf.at[b], isem)
            pltpu.async_copy(dst_hbm.at[pl.ds(off, CHUNK)], dbuf.at[b], isem)

        def idx_wait(i, b):
            off = ebase + i * CHUNK
            pltpu.make_async_copy(
                src_hbm.at[pl.ds(off, CHUNK)], sbuf.at[b], isem).wait()
            pltpu.make_async_copy(
                dst_hbm.at[pl.ds(off, CHUNK)], dbuf.at[b], isem).wait()

        def g_issue(b):
            pltpu.async_copy(h_hbm.at[sbuf.at[b]], rows[b], gsem)

        def g_wait(b):
            pltpu.make_async_copy(h_hbm.at[sbuf.at[b]], rows[b], gsem).wait()

        def s_issue(b):
            pltpu.async_copy(rows[b], acc_sh.at[dbuf.at[b]], ssem, add=True)

        def s_wait(b):
            pltpu.make_async_copy(rows[b], acc_sh.at[dbuf.at[b]], ssem).wait()

        for b in range(NBUF):
            idx_load(b, b)

        zero16 = jnp.zeros((16,), jnp.float32)

        def zfill(i, carry):
            rows[0][i // LANES, pl.ds((i % LANES) * 16, 16)] = zero16
            return carry

        lax.fori_loop(0, ZR * LANES, zfill, 0)

        r0 = s * RPT

        def zcopy(j, carry):
            pltpu.async_copy(rows[0], acc_sh.at[pl.ds(r0 + j * ZR, ZR)], ssem)
            return carry

        lax.fori_loop(0, NZ, zcopy, 0)

        def zdrain(j, carry):
            pltpu.make_async_copy(
                rows[0], acc_sh.at[pl.ds(r0 + j * ZR, ZR)], ssem).wait()
            return carry

        lax.fori_loop(0, NZ, zdrain, 0)
        plsc.subcore_barrier()

        for b in range(NBUF):
            idx_wait(b, b)
            g_issue(b)

        def body(k, carry):
            i0 = NBUF * k
            for b in range(NBUF):
                g_wait(b)
                s_issue(b)
            for b in range(NBUF):
                i = i0 + b
                inext = i + NBUF
                s_wait(b)

                @pl.when(inext < NITER)
                def _():
                    idx_load(inext, b)
                    idx_wait(inext, b)
                    g_issue(b)

            return carry

        lax.fori_loop(0, NQUAD, body, 0)
        g_wait(0)
        s_issue(0)
        s_wait(0)
        plsc.subcore_barrier()

        def ocopy(j, carry):
            sl = pl.ds(r0 + j * ZR, ZR)
            pltpu.async_copy(acc_sh.at[sl], out_hbm.at[c, sl], ssem)
            return carry

        lax.fori_loop(0, NZ, ocopy, 0)

        def odrain(j, carry):
            sl = pl.ds(r0 + j * ZR, ZR)
            pltpu.make_async_copy(acc_sh.at[sl], out_hbm.at[c, sl], ssem).wait()
            return carry

        lax.fori_loop(0, NZ, odrain, 0)

    return agg


_agg = _make_agg()


BLK = 2000


def _mlp_body(h_ref, p0_ref, p1_ref, w1_ref, b1_ref, w2_ref, b2_ref,
              sc_ref, sh_ref, o_ref):
    z = h_ref[...] + p0_ref[...] + p1_ref[...]
    z = jnp.dot(z, w1_ref[...], preferred_element_type=jnp.float32)
    z = jnp.maximum(z + b1_ref[...], 0.0)
    z = jnp.dot(z, w2_ref[...], preferred_element_type=jnp.float32)
    z = jnp.maximum(z + b2_ref[...], 0.0)
    o_ref[...] = jnp.maximum(z * sc_ref[...] + sh_ref[...], 0.0)


_row_spec = pl.BlockSpec((BLK, D), lambda i: (i, 0))
_w_spec = pl.BlockSpec((D, D), lambda i: (0, 0))
_v_spec = pl.BlockSpec((1, D), lambda i: (0, 0))

_mlp = pl.pallas_call(
    _mlp_body,
    grid=(N // BLK,),
    in_specs=[_row_spec, _row_spec, _row_spec,
              _w_spec, _v_spec, _w_spec, _v_spec, _v_spec, _v_spec],
    out_specs=_row_spec,
    out_shape=jax.ShapeDtypeStruct((N, D), jnp.float32),
)


def kernel(x, edge_index,
           l0_W1, l0_b1, l0_W2, l0_b2, l0_gamma, l0_beta, l0_rm, l0_rv,
           l1_W1, l1_b1, l1_W2, l1_b2, l1_gamma, l1_beta, l1_rm, l1_rv,
           l2_W1, l2_b1, l2_W2, l2_b2, l2_gamma, l2_beta, l2_rm, l2_rv):
    src = edge_index[0]
    dst = edge_index[1]
    params = [
        (l0_W1, l0_b1, l0_W2, l0_b2, l0_gamma, l0_beta, l0_rm, l0_rv),
        (l1_W1, l1_b1, l1_W2, l1_b2, l1_gamma, l1_beta, l1_rm, l1_rv),
        (l2_W1, l2_b1, l2_W2, l2_b2, l2_gamma, l2_beta, l2_rm, l2_rv),
    ]
    h = x
    for (W1, b1, W2, b2, gamma, beta, rm, rv) in params:
        p = _agg(h, src, dst)[:, :N, :]
        scale = gamma * lax.rsqrt(rv + BN_EPS)
        shift = beta - rm * scale
        h = _mlp(h, p[0], p[1],
                 W1, b1.reshape(1, D), W2, b2.reshape(1, D),
                 scale.reshape(1, D), shift.reshape(1, D))
    return h


# speedup vs baseline: 12.1443x; 1.0512x over previous
import functools

import jax
import jax.numpy as jnp
from jax import lax
from jax.experimental import pallas as pl
from jax.experimental.pallas import tpu as pltpu
from jax.experimental.pallas import tpu_sc as plsc

N = 10000
E = 320000
D = 128
BN_EPS = 1e-5

NC = 2
NS = 16
NW = NC * NS
EPT = E // NW
CHUNK = 80
NITER = EPT // CHUNK
NBUF = 4
NQUAD = (NITER - 1) // NBUF
IBYTES = CHUNK * 4
NP = 10240
RPT = NP // NS
ZR = CHUNK
NZ = RPT // ZR
LANES = D // 16


def _make_agg():
    mesh = plsc.VectorSubcoreMesh(core_axis_name="c", subcore_axis_name="s")

    @functools.partial(
        pl.kernel,
        out_type=jax.ShapeDtypeStruct((NC, NP, D), jnp.float32),
        mesh=mesh,
        scratch_types=[
            pltpu.VMEM((NBUF, CHUNK), jnp.int32),
            pltpu.VMEM((NBUF, CHUNK), jnp.int32),
            [pltpu.VMEM((CHUNK, D), jnp.float32) for _ in range(NBUF)],
            pltpu.VMEM_SHARED((NP, D), jnp.float32),
            pltpu.SemaphoreType.DMA,
            pltpu.SemaphoreType.DMA,
            pltpu.SemaphoreType.DMA,
        ],
    )
    def agg(h_hbm, src_hbm, dst_hbm, out_hbm, sbuf, dbuf, rows,
            acc_sh, isem, gsem, ssem):
        c = lax.axis_index("c")
        s = lax.axis_index("s")
        wid = c * NS + s
        ebase = wid * EPT

        def idx_load(i, b):
            off = ebase + i * CHUNK
            pltpu.async_copy(src_hbm.at[pl.ds(off, CHUNK)], sbuf.at[b], isem)
            pltpu.async_copy(dst_hbm.at[pl.ds(off, CHUNK)], dbuf.at[b], isem)

        def idx_wait(i, b):
            off = ebase + i * CHUNK
            pltpu.make_async_copy(
                src_hbm.at[pl.ds(off, CHUNK)], sbuf.at[b], isem).wait()
            pltpu.make_async_copy(
                dst_hbm.at[pl.ds(off, CHUNK)], dbuf.at[b], isem).wait()

        def g_issue(b):
            pltpu.async_copy(h_hbm.at[sbuf.at[b]], rows[b], gsem)

        def g_wait(b):
            pltpu.make_async_copy(h_hbm.at[sbuf.at[b]], rows[b], gsem).wait()

        def s_issue(b):
            pltpu.async_copy(rows[b], acc_sh.at[dbuf.at[b]], ssem, add=True)

        def s_wait(b):
            pltpu.make_async_copy(rows[b], acc_sh.at[dbuf.at[b]], ssem).wait()

        for b in range(NBUF):
            idx_load(b, b)

        zero16 = jnp.zeros((16,), jnp.float32)

        def zfill(i, carry):
            rows[0][i // LANES, pl.ds((i % LANES) * 16, 16)] = zero16
            return carry

        lax.fori_loop(0, ZR * LANES, zfill, 0)

        r0 = s * RPT

        def zcopy(j, carry):
            pltpu.async_copy(rows[0], acc_sh.at[pl.ds(r0 + j * ZR, ZR)], ssem)
            return carry

        lax.fori_loop(0, NZ, zcopy, 0)

        def zdrain(j, carry):
            pltpu.make_async_copy(
                rows[0], acc_sh.at[pl.ds(r0 + j * ZR, ZR)], ssem).wait()
            return carry

        lax.fori_loop(0, NZ, zdrain, 0)
        plsc.subcore_barrier()

        for b in range(NBUF):
            idx_wait(b, b)
            g_issue(b)

        def body(k, carry):
            i0 = NBUF * k
            for b in range(NBUF):
                g_wait(b)
                s_issue(b)
            for b in range(NBUF):
                i = i0 + b
                inext = i + NBUF
                s_wait(b)

                @pl.when(inext < NITER)
                def _():
                    idx_load(inext, b)
                    idx_wait(inext, b)
                    g_issue(b)

            return carry

        lax.fori_loop(0, NQUAD, body, 0)
        g_wait(0)
        s_issue(0)
        s_wait(0)
        plsc.subcore_barrier()

        def ocopy(j, carry):
            sl = pl.ds(r0 + j * ZR, ZR)
            pltpu.async_copy(acc_sh.at[sl], out_hbm.at[c, sl], ssem)
            return carry

        lax.fori_loop(0, NZ, ocopy, 0)

        def odrain(j, carry):
            sl = pl.ds(r0 + j * ZR, ZR)
            pltpu.make_async_copy(acc_sh.at[sl], out_hbm.at[c, sl], ssem).wait()
            return carry

        lax.fori_loop(0, NZ, odrain, 0)

    return agg


_agg = _make_agg()


BLK = 2000


def _mlp_body(h_ref, p_ref, w1_ref, b1_ref, w2_ref, b2_ref,
              sc_ref, sh_ref, o_ref):
    z = h_ref[...] + p_ref[0] + p_ref[1]
    z = jnp.dot(z, w1_ref[...], preferred_element_type=jnp.float32)
    z = jnp.maximum(z + b1_ref[...], 0.0)
    z = jnp.dot(z, w2_ref[...], preferred_element_type=jnp.float32)
    z = jnp.maximum(z + b2_ref[...], 0.0)
    o_ref[...] = jnp.maximum(z * sc_ref[...] + sh_ref[...], 0.0)


_row_spec = pl.BlockSpec((BLK, D), lambda i: (i, 0))
_p_spec = pl.BlockSpec((2, BLK, D), lambda i: (0, i, 0))
_w_spec = pl.BlockSpec((D, D), lambda i: (0, 0))
_v_spec = pl.BlockSpec((1, D), lambda i: (0, 0))

_mlp = pl.pallas_call(
    _mlp_body,
    grid=(N // BLK,),
    in_specs=[_row_spec, _p_spec,
              _w_spec, _v_spec, _w_spec, _v_spec, _v_spec, _v_spec],
    out_specs=_row_spec,
    out_shape=jax.ShapeDtypeStruct((N, D), jnp.float32),
)


def kernel(x, edge_index,
           l0_W1, l0_b1, l0_W2, l0_b2, l0_gamma, l0_beta, l0_rm, l0_rv,
           l1_W1, l1_b1, l1_W2, l1_b2, l1_gamma, l1_beta, l1_rm, l1_rv,
           l2_W1, l2_b1, l2_W2, l2_b2, l2_gamma, l2_beta, l2_rm, l2_rv):
    src = edge_index[0]
    dst = edge_index[1]
    params = [
        (l0_W1, l0_b1, l0_W2, l0_b2, l0_gamma, l0_beta, l0_rm, l0_rv),
        (l1_W1, l1_b1, l1_W2, l1_b2, l1_gamma, l1_beta, l1_rm, l1_rv),
        (l2_W1, l2_b1, l2_W2, l2_b2, l2_gamma, l2_beta, l2_rm, l2_rv),
    ]
    prepped = []
    for (W1, b1, W2, b2, gamma, beta, rm, rv) in params:
        scale = gamma * lax.rsqrt(rv + BN_EPS)
        shift = beta - rm * scale
        prepped.append((W1, b1.reshape(1, D), W2, b2.reshape(1, D),
                        scale.reshape(1, D), shift.reshape(1, D)))

    h = x
    for (W1, b1, W2, b2, scale, shift) in prepped:
        p = _agg(h, src, dst)
        h = _mlp(h, p, W1, b1, W2, b2, scale, shift)
    return h
